# baseline (speedup 1.0000x reference)
import functools

import jax
import jax.numpy as jnp
from jax import lax
from jax.experimental import pallas as pl
from jax.experimental.pallas import tpu as pltpu
from jax.experimental.pallas import tpu_sc as plsc

S = 34
H = 2048
W = 1024
N = H * W
NW = 32
PER_W = N // NW
C = 1024
NCHUNK = PER_W // C
L = 16


def _sc_body(x_hbm, xt_hbm, xn_hbm, xmin_hbm, xmax_hbm, xto_hbm,
             xa, xta, xmin_v, xmax_v, xto_v):
    cid = lax.axis_index("c")
    sid = lax.axis_index("s")
    wid = sid * 2 + cid
    base0 = wid * PER_W

    def chunk_body(ci, _):
        base = base0 + ci * C
        pltpu.sync_copy(x_hbm.at[:, pl.ds(base, C)], xa)
        pltpu.sync_copy(xt_hbm.at[pl.ds(base, C)], xta)

        def vec_body(v, _):
            sl = pl.ds(v * L, L)
            def acc_body(s, acc):
                return acc + jnp.abs(xa[s, sl])
            asum = lax.fori_loop(1, S, acc_body, jnp.zeros((L,), jnp.float32),
                                 unroll=True)
            x0 = xa[0, sl]
            xmin = x0 - asum
            xmax = x0 + asum
            sgn = jnp.sign(xmin) + jnp.sign(xmax)
            mask_p = sgn == 0.0
            mask_0 = sgn < 0.0
            denom = jnp.abs(xmax) + jnp.abs(xmin)
            coef = jnp.where(denom > 0.0, xmax / jnp.where(denom > 0.0, denom, 1.0), 0.0)
            bias = xmax * (1.0 - coef) * 0.5
            noise = jnp.abs(bias)
            out0 = jnp.where(mask_p, coef * x0 + bias,
                             jnp.where(mask_0, 0.0, x0))
            xa[0, sl] = out0

            def mid_body(s, ns_acc):
                xm = xa[s, sl]
                om = jnp.where(mask_p, coef * xm, jnp.where(mask_0, 0.0, xm))
                xa[s, sl] = om
                return ns_acc + jnp.abs(om)
            nsum = lax.fori_loop(1, S - 1, mid_body,
                                 jnp.zeros((L,), jnp.float32), unroll=True)

            xl = xa[S - 1, sl]
            ol = jnp.where(mask_p, noise + jnp.abs(coef) * xl,
                           jnp.where(mask_0, 0.0, xl))
            xa[S - 1, sl] = ol
            nsum = nsum + jnp.abs(ol)
            xmin_v[sl] = out0 - nsum
            xmax_v[sl] = out0 + nsum
            xto_v[sl] = jnp.maximum(xta[sl], 0.0)
            return 0

        lax.fori_loop(0, C // L, vec_body, 0)

        pltpu.sync_copy(xa, xn_hbm.at[:, pl.ds(base, C)])
        pltpu.sync_copy(xmin_v, xmin_hbm.at[pl.ds(base, C)])
        pltpu.sync_copy(xmax_v, xmax_hbm.at[pl.ds(base, C)])
        pltpu.sync_copy(xto_v, xto_hbm.at[pl.ds(base, C)])
        return 0

    lax.fori_loop(0, NCHUNK, chunk_body, 0)


@jax.jit
def _run(x2, xt):
    mesh = plsc.VectorSubcoreMesh(core_axis_name="c", subcore_axis_name="s")
    f = pl.kernel(
        _sc_body,
        out_type=(
            jax.ShapeDtypeStruct((S, N), jnp.float32),
            jax.ShapeDtypeStruct((N,), jnp.float32),
            jax.ShapeDtypeStruct((N,), jnp.float32),
            jax.ShapeDtypeStruct((N,), jnp.float32),
        ),
        mesh=mesh,
        scratch_types=[
            pltpu.VMEM((S, C), jnp.float32),
            pltpu.VMEM((C,), jnp.float32),
            pltpu.VMEM((C,), jnp.float32),
            pltpu.VMEM((C,), jnp.float32),
            pltpu.VMEM((C,), jnp.float32),
        ],
    )
    return f(x2, xt)


def kernel(x, x_min, x_max, x_true):
    del x_min, x_max
    x2 = x.reshape(S, N)
    xt = x_true.reshape(N)
    xn, xmin2, xmax2, xto = _run(x2, xt)
    return (xn.reshape(S, H, W), xmin2.reshape(H, W), xmax2.reshape(H, W),
            xto.reshape(H, W))
